# baseline (speedup 1.0000x reference)
import functools

import jax
import jax.numpy as jnp
from jax import lax
from jax.experimental import pallas as pl
from jax.experimental.pallas import tpu as pltpu
from jax.experimental.pallas import tpu_sc as plsc

D = 64
G = 128
NBUF = 4


@functools.cache
def _make_kernel(B):
    info = plsc.get_sparse_core_info()
    NC, NS = info.num_cores, info.num_subcores
    NW = NC * NS
    assert B % (G * NW) == 0
    n_chunks = B // (G * NW)
    assert n_chunks % NBUF == 0
    mesh = plsc.VectorSubcoreMesh(core_axis_name="c", subcore_axis_name="s")

    @functools.partial(
        pl.kernel,
        out_type=jax.ShapeDtypeStruct((B, D), jnp.float32),
        mesh=mesh,
        scratch_types=(
            [pltpu.VMEM((n_chunks, G), jnp.int32)]
            + [pltpu.VMEM((G, D), jnp.float32) for _ in range(NBUF)]
            + [pltpu.SemaphoreType.DMA for _ in range(NBUF)]
        ),
        compiler_params=pltpu.CompilerParams(use_tc_tiling_on_sc=False),
    )
    def k(idx_hbm, table_hbm, out_hbm, idx_v, *bufs_and_sems):
        bufs = bufs_and_sems[:NBUF]
        sems = bufs_and_sems[NBUF:]
        wid = lax.axis_index("s") * NC + lax.axis_index("c")
        cbase = wid * n_chunks
        pltpu.sync_copy(idx_hbm.at[pl.ds(cbase, n_chunks)], idx_v)

        def outer(t, carry):
            g0 = t * NBUF
            for b in range(NBUF):
                pltpu.async_copy(
                    table_hbm.at[idx_v.at[g0 + b]], bufs[b], sems[b]
                )
            for b in range(NBUF):
                pltpu.make_async_copy(
                    table_hbm.at[idx_v.at[g0 + b]], bufs[b], sems[b]
                ).wait()
                pltpu.sync_copy(
                    bufs[b], out_hbm.at[pl.ds((cbase + g0 + b) * G, G)]
                )
            return carry

        lax.fori_loop(0, n_chunks // NBUF, outer, 0)

    return k


def kernel(x, table):
    B = x.size
    xf = x.reshape(B // G, G)
    out = _make_kernel(B)(xf, table)
    return out.reshape(x.shape + (table.shape[1],))


# speedup vs baseline: 1.0105x; 1.0105x over previous
import functools

import jax
import jax.numpy as jnp
from jax import lax
from jax.experimental import pallas as pl
from jax.experimental.pallas import tpu as pltpu
from jax.experimental.pallas import tpu_sc as plsc

D = 64
G = 320
NBUF = 4


@functools.cache
def _make_kernel(B):
    info = plsc.get_sparse_core_info()
    NC, NS = info.num_cores, info.num_subcores
    NW = NC * NS
    assert B % (G * NW) == 0
    n_chunks = B // (G * NW)
    assert n_chunks % NBUF == 0
    mesh = plsc.VectorSubcoreMesh(core_axis_name="c", subcore_axis_name="s")

    @functools.partial(
        pl.kernel,
        out_type=jax.ShapeDtypeStruct((B, D), jnp.float32),
        mesh=mesh,
        scratch_types=(
            [pltpu.VMEM((n_chunks, G), jnp.int32)]
            + [pltpu.VMEM((G, D), jnp.float32) for _ in range(NBUF)]
            + [pltpu.SemaphoreType.DMA for _ in range(NBUF)]
        ),
        compiler_params=pltpu.CompilerParams(use_tc_tiling_on_sc=False),
    )
    def k(idx_hbm, table_hbm, out_hbm, idx_v, *bufs_and_sems):
        bufs = bufs_and_sems[:NBUF]
        sems = bufs_and_sems[NBUF:]
        wid = lax.axis_index("s") * NC + lax.axis_index("c")
        cbase = wid * n_chunks
        pltpu.sync_copy(idx_hbm.at[pl.ds(cbase, n_chunks)], idx_v)

        def outer(t, carry):
            g0 = t * NBUF
            for b in range(NBUF):
                pltpu.async_copy(
                    table_hbm.at[idx_v.at[g0 + b]], bufs[b], sems[b]
                )
            for b in range(NBUF):
                pltpu.make_async_copy(
                    table_hbm.at[idx_v.at[g0 + b]], bufs[b], sems[b]
                ).wait()
                pltpu.sync_copy(
                    bufs[b], out_hbm.at[pl.ds((cbase + g0 + b) * G, G)]
                )
            return carry

        lax.fori_loop(0, n_chunks // NBUF, outer, 0)

    return k


def kernel(x, table):
    B = x.size
    xf = x.reshape(B // G, G)
    out = _make_kernel(B)(xf, table)
    return out.reshape(x.shape + (table.shape[1],))
